# baseline (speedup 1.0000x reference)
import functools

import jax
import jax.numpy as jnp
from jax import lax
from jax.experimental import pallas as pl
from jax.experimental.pallas import tpu as pltpu
from jax.experimental.pallas import tpu_sc as plsc

K = 8192
C = 64
BETA_ = 0.25

NC, NS = 2, 16
NW = NC * NS


_TN = 512


def _argmax_body(f_ref, w_ref, idx_ref, wn_ref):
    @pl.when(pl.program_id(0) == 0)
    def _():
        w = w_ref[...]
        n = jnp.sqrt(jnp.sum(w * w, axis=1, keepdims=True))
        wn_ref[...] = w / jnp.maximum(n, 1e-12)

    f = f_ref[...]
    fn = jnp.sqrt(jnp.sum(f * f, axis=1, keepdims=True))
    f = f / jnp.maximum(fn, 1e-12)
    wn = wn_ref[...]
    s = lax.dot_general(f, wn, (((1,), (1,)), ((), ())),
                        preferred_element_type=jnp.float32)
    idx = jnp.argmax(s, axis=1)
    idx_ref[0, 0, :] = idx.astype(jnp.int32)


def _argmax_call(f_NxC, W, n_tokens):
    nb = n_tokens // _TN
    out = pl.pallas_call(
        _argmax_body,
        grid=(nb,),
        in_specs=[
            pl.BlockSpec((_TN, C), lambda i: (i, 0)),
            pl.BlockSpec((K, C), lambda i: (0, 0)),
        ],
        out_specs=pl.BlockSpec((1, 1, _TN), lambda i: (i, 0, 0)),
        out_shape=jax.ShapeDtypeStruct((nb, 1, _TN), jnp.int32),
        scratch_shapes=[pltpu.VMEM((K, C), jnp.float32)],
    )(f_NxC, W)
    return out.reshape(n_tokens)


def _sc_body(w_hbm, idx_hbm, ones_hbm, zeros_hbm, fhat_hbm, counts_hbm,
             idx_v, rows_v, ones_v, cnt_sh, sem_g, sem_z, sem_o, sem_w, bpw):
    cid = lax.axis_index("c")
    sid = lax.axis_index("s")
    wid = sid * NC + cid
    base = wid * bpw
    rows_per_tile = K // NS

    zcp = pltpu.async_copy(zeros_hbm,
                           cnt_sh.at[pl.ds(sid * rows_per_tile,
                                           rows_per_tile)], sem_z)
    ocp = pltpu.async_copy(ones_hbm, ones_v, sem_o)

    pltpu.sync_copy(idx_hbm.at[pl.ds(base, bpw)], idx_v)
    gcp = pltpu.async_copy(w_hbm.at[idx_v], rows_v, sem_g)

    zcp.wait()
    ocp.wait()
    plsc.subcore_barrier()

    gcp.wait()
    wcp = pltpu.async_copy(rows_v, fhat_hbm.at[pl.ds(base, bpw)], sem_w)

    pltpu.sync_copy(ones_v, cnt_sh.at[idx_v], add=True)
    plsc.subcore_barrier()

    pltpu.sync_copy(cnt_sh.at[pl.ds(sid * rows_per_tile, rows_per_tile)],
                    counts_hbm.at[cid, pl.ds(sid * rows_per_tile,
                                             rows_per_tile)])
    wcp.wait()


def _sc_gather_counts(W, idx, n_tokens):
    bpw = n_tokens // NW
    rows_per_tile = K // NS
    mesh = plsc.VectorSubcoreMesh(core_axis_name="c", subcore_axis_name="s")
    fn = pl.kernel(
        functools.partial(_sc_body, bpw=bpw),
        out_type=(jax.ShapeDtypeStruct((n_tokens, C), jnp.float32),
                  jax.ShapeDtypeStruct((NC, K, 16), jnp.float32)),
        mesh=mesh,
        scratch_types=[
            pltpu.VMEM((bpw,), jnp.int32),
            pltpu.VMEM((bpw, C), jnp.float32),
            pltpu.VMEM((bpw, 16), jnp.float32),
            pltpu.VMEM_SHARED((K, 16), jnp.float32),
            pltpu.SemaphoreType.DMA,
            pltpu.SemaphoreType.DMA,
            pltpu.SemaphoreType.DMA,
            pltpu.SemaphoreType.DMA,
        ],
        compiler_params=pltpu.CompilerParams(use_tc_tiling_on_sc=False),
    )
    ones = jnp.ones((bpw, 16), jnp.float32)
    zeros = jnp.zeros((rows_per_tile, 16), jnp.float32)
    return fn(W, idx, ones, zeros)


def _loss_body(fhat_ref, f_ref, cnt_ref, fhatT_ref, vq_ref, use_ref,
               n_elems):
    i = pl.program_id(0)
    fhatT = jnp.transpose(fhat_ref[0], (1, 0))
    fhatT_ref[0] = fhatT
    d = fhatT - f_ref[0]
    part = jnp.sum(d * d) * ((1.0 + BETA_) / n_elems)

    @pl.when(i == 0)
    def _():
        vq_ref[0, 0] = 0.0
        c = cnt_ref[0] + cnt_ref[1]
        use_ref[0, 0] = jnp.mean((c > 0.0).astype(jnp.float32)) * 100.0

    vq_ref[0, 0] += part


def _losses(f3, fhat_B_HW_C, counts):
    b, c, hw = f3.shape
    n_elems = b * c * hw
    return pl.pallas_call(
        functools.partial(_loss_body, n_elems=n_elems),
        grid=(b,),
        in_specs=[
            pl.BlockSpec((1, hw, c), lambda i: (i, 0, 0)),
            pl.BlockSpec((1, c, hw), lambda i: (i, 0, 0)),
            pl.BlockSpec((NC, K, 16), lambda i: (0, 0, 0)),
        ],
        out_specs=(pl.BlockSpec((1, c, hw), lambda i: (i, 0, 0)),
                   pl.BlockSpec(memory_space=pltpu.SMEM),
                   pl.BlockSpec(memory_space=pltpu.SMEM)),
        out_shape=(jax.ShapeDtypeStruct((b, c, hw), jnp.float32),
                   jax.ShapeDtypeStruct((1, 1), jnp.float32),
                   jax.ShapeDtypeStruct((1, 1), jnp.float32)),
    )(fhat_B_HW_C, f3, counts)


def kernel(f_BChw, W):
    f_BChw = f_BChw.astype(jnp.float32)
    B, Cc, h, w = f_BChw.shape
    n_tokens = B * h * w
    f3 = f_BChw.reshape(B, Cc, h * w)
    f_NxC = f_BChw.transpose(0, 2, 3, 1).reshape(n_tokens, Cc)

    idx = _argmax_call(f_NxC, W, n_tokens)
    fhat_NxC, counts = _sc_gather_counts(W, idx, n_tokens)
    fhatT, vq, use = _losses(f3, fhat_NxC.reshape(B, h * w, Cc), counts)

    fhat_BChw = fhatT.reshape(B, Cc, h, w)
    return (fhat_BChw, vq[0, 0], jnp.float32(0.0), use[0, 0])


# speedup vs baseline: 1.2241x; 1.2241x over previous
import functools

import jax
import jax.numpy as jnp
from jax import lax
from jax.experimental import pallas as pl
from jax.experimental.pallas import tpu as pltpu
from jax.experimental.pallas import tpu_sc as plsc

K = 8192
C = 64
BETA_ = 0.25

NC, NS = 2, 16
NW = NC * NS


_TN = 512


def _argmax_body(f_ref, w_ref, idx_ref, wn_ref):
    @pl.when(pl.program_id(0) == 0)
    def _():
        w = w_ref[...]
        n = jnp.sqrt(jnp.sum(w * w, axis=1, keepdims=True))
        wn_ref[...] = w / jnp.maximum(n, 1e-12)

    f = f_ref[...]
    fn = jnp.sqrt(jnp.sum(f * f, axis=1, keepdims=True))
    f = f / jnp.maximum(fn, 1e-12)
    wn = wn_ref[...]
    s = lax.dot_general(f, wn, (((1,), (1,)), ((), ())),
                        preferred_element_type=jnp.float32)
    idx = jnp.argmax(s, axis=1)
    idx_ref[0, 0, :] = idx.astype(jnp.int32)


def _argmax_call(f_NxC, W, n_tokens):
    nb = n_tokens // _TN
    out = pl.pallas_call(
        _argmax_body,
        grid=(nb,),
        in_specs=[
            pl.BlockSpec((_TN, C), lambda i: (i, 0)),
            pl.BlockSpec((K, C), lambda i: (0, 0)),
        ],
        out_specs=pl.BlockSpec((1, 1, _TN), lambda i: (i, 0, 0)),
        out_shape=jax.ShapeDtypeStruct((nb, 1, _TN), jnp.int32),
        scratch_shapes=[pltpu.VMEM((K, C), jnp.float32)],
    )(f_NxC, W)
    return out.reshape(n_tokens)


def _sc_body(w_hbm, idx_hbm, ones_hbm, zeros_hbm, fhat_hbm, counts_hbm,
             idx_v, rows_v, ones_v, cnt_sh, sem_g, sem_z, sem_o, sem_w, bpw):
    cid = lax.axis_index("c")
    sid = lax.axis_index("s")
    wid = sid * NC + cid
    base = wid * bpw
    rows_per_tile = K // NS

    zcp = pltpu.async_copy(zeros_hbm,
                           cnt_sh.at[pl.ds(sid * rows_per_tile,
                                           rows_per_tile)], sem_z)
    ocp = pltpu.async_copy(ones_hbm, ones_v, sem_o)

    pltpu.sync_copy(idx_hbm.at[pl.ds(base, bpw)], idx_v)
    gcp = pltpu.async_copy(w_hbm.at[idx_v], rows_v, sem_g)

    zcp.wait()
    ocp.wait()
    plsc.subcore_barrier()

    gcp.wait()
    wcp = pltpu.async_copy(rows_v, fhat_hbm.at[pl.ds(base, bpw)], sem_w)

    pltpu.sync_copy(ones_v, cnt_sh.at[idx_v], add=True)
    plsc.subcore_barrier()

    pltpu.sync_copy(cnt_sh.at[pl.ds(sid * rows_per_tile, rows_per_tile)],
                    counts_hbm.at[cid, pl.ds(sid * rows_per_tile,
                                             rows_per_tile)])
    wcp.wait()


def _sc_gather_counts(W, idx, n_tokens):
    bpw = n_tokens // NW
    rows_per_tile = K // NS
    mesh = plsc.VectorSubcoreMesh(core_axis_name="c", subcore_axis_name="s")
    fn = pl.kernel(
        functools.partial(_sc_body, bpw=bpw),
        out_type=(jax.ShapeDtypeStruct((n_tokens, C), jnp.float32),
                  jax.ShapeDtypeStruct((NC, K, 16), jnp.float32)),
        mesh=mesh,
        scratch_types=[
            pltpu.VMEM((bpw,), jnp.int32),
            pltpu.VMEM((bpw, C), jnp.float32),
            pltpu.VMEM((bpw, 16), jnp.float32),
            pltpu.VMEM_SHARED((K, 16), jnp.float32),
            pltpu.SemaphoreType.DMA,
            pltpu.SemaphoreType.DMA,
            pltpu.SemaphoreType.DMA,
            pltpu.SemaphoreType.DMA,
        ],
        compiler_params=pltpu.CompilerParams(use_tc_tiling_on_sc=False),
    )
    ones = jnp.ones((bpw, 16), jnp.float32)
    zeros = jnp.zeros((rows_per_tile, 16), jnp.float32)
    return fn(W, idx, ones, zeros)


def _loss_body(f_ref, fhat_ref, cnt_ref, vq_ref, use_ref):
    d = fhat_ref[...] - f_ref[...]
    vq_ref[0, 0] = (1.0 + BETA_) * jnp.mean(d * d)
    c = cnt_ref[pl.ds(0, K * 16)] + cnt_ref[pl.ds(K * 16, K * 16)]
    flags = jnp.where(c > 0.0, 1.0, 0.0)
    use_ref[0, 0] = jnp.sum(flags) * (100.0 / (16.0 * K))


def _losses(f_BhwC, fhat_BhwC, counts_flat):
    return pl.pallas_call(
        _loss_body,
        out_specs=(pl.BlockSpec(memory_space=pltpu.SMEM),
                   pl.BlockSpec(memory_space=pltpu.SMEM)),
        out_shape=(jax.ShapeDtypeStruct((1, 1), jnp.float32),
                   jax.ShapeDtypeStruct((1, 1), jnp.float32)),
    )(f_BhwC, fhat_BhwC, counts_flat)


def kernel(f_BChw, W):
    f_BChw = f_BChw.astype(jnp.float32)
    B, Cc, h, w = f_BChw.shape
    n_tokens = B * h * w
    f_BhwC = f_BChw.transpose(0, 2, 3, 1)
    f_NxC = f_BhwC.reshape(n_tokens, Cc)

    idx = _argmax_call(f_NxC, W, n_tokens)
    fhat_NxC, counts = _sc_gather_counts(W, idx, n_tokens)
    fhat_BhwC = fhat_NxC.reshape(B, h, w, Cc)
    vq, use = _losses(f_BhwC, fhat_BhwC, counts.reshape(NC * K * 16))

    fhat_BChw = fhat_BhwC.transpose(0, 3, 1, 2)
    return (fhat_BChw, vq[0, 0], jnp.float32(0.0), use[0, 0])
